# baseline (speedup 1.0000x reference)
import math

import jax
import jax.numpy as jnp
from jax import lax
from jax.experimental import pallas as pl
from jax.experimental.pallas import tpu as pltpu
from jax.experimental.pallas import tpu_sc as plsc

_SEQ, _BATCH, _EMB = 200, 1024, 128
_N = _SEQ * _BATCH
_NC, _NS, _L = 2, 16, 16
_NW = _NC * _NS
_PER_W = _N // _NW
_CHUNK = 128
_NCHUNK = _PER_W // _CHUNK
_NBUF = 4
_ROWS_W = 16
_SCALE = math.sqrt(_EMB)


def _body(tok_hbm, table_hbm, out_hbm, idx_v, *rest):
    bufs = rest[:_NBUF]
    gsems = rest[_NBUF:2 * _NBUF]
    ssems = rest[2 * _NBUF:3 * _NBUF]
    wid = lax.axis_index("s") * _NC + lax.axis_index("c")
    base = wid * _PER_W
    r0 = base // _BATCH
    r0a = pl.multiple_of(jnp.minimum(r0 - r0 % 8, _SEQ - _ROWS_W), 8)
    off0 = base - r0a * _BATCH
    pltpu.sync_copy(tok_hbm.at[pl.ds(r0a, _ROWS_W)], idx_v)

    def start_gather(step, b):
        off = off0 + step * _CHUNK
        idx = idx_v.at[off // _BATCH, pl.ds(off % _BATCH, _CHUNK)]
        pltpu.async_copy(table_hbm.at[idx], bufs[b], gsems[b])

    def scale(buf):
        def _mul_row(i, _):
            for k in range(_EMB // _L):
                sl = (i, pl.ds(k * _L, _L))
                buf[sl] = buf[sl] * _SCALE
            return 0

        lax.fori_loop(0, _CHUNK, _mul_row, 0)

    for j in range(_NBUF - 1):
        start_gather(j, j)
    scat = [None] * _NBUF
    for j in range(_NCHUNK):
        b = j % _NBUF
        ahead = j + _NBUF - 1
        if ahead < _NCHUNK:
            nb = ahead % _NBUF
            if scat[nb] is not None:
                scat[nb].wait()
            start_gather(ahead, nb)
        pltpu.make_async_copy(
            table_hbm.at[pl.ds(0, _CHUNK)], bufs[b], gsems[b]
        ).wait()
        scale(bufs[b])
        scat[b] = pltpu.async_copy(
            bufs[b], out_hbm.at[pl.ds(base + j * _CHUNK, _CHUNK)], ssems[b]
        )
    for h in scat:
        if h is not None:
            h.wait()


@jax.jit
def kernel(tokens, table):
    tok = tokens.astype(jnp.int32)
    mesh = plsc.VectorSubcoreMesh(core_axis_name="c", subcore_axis_name="s")
    out = pl.kernel(
        _body,
        out_type=jax.ShapeDtypeStruct((_N, _EMB), jnp.float32),
        mesh=mesh,
        scratch_types=(
            [pltpu.VMEM((_ROWS_W, _BATCH), jnp.int32)]
            + [pltpu.VMEM((_CHUNK, _EMB), jnp.float32) for _ in range(_NBUF)]
            + [pltpu.SemaphoreType.DMA for _ in range(2 * _NBUF)]
        ),
    )(tok, table)
    return out.reshape(_SEQ, _BATCH, _EMB)


# speedup vs baseline: 1.0291x; 1.0291x over previous
import math

import jax
import jax.numpy as jnp
from jax import lax
from jax.experimental import pallas as pl
from jax.experimental.pallas import tpu as pltpu
from jax.experimental.pallas import tpu_sc as plsc

_SEQ, _BATCH, _EMB = 200, 1024, 128
_N = _SEQ * _BATCH
_NC, _NS, _L = 2, 16, 16
_NW = _NC * _NS
_PER_W = _N // _NW
_CHUNK = 128
_NCHUNK = _PER_W // _CHUNK
_NBUF = 6
_SCALE = math.sqrt(_EMB)


def _body(tok_hbm, table_hbm, out_hbm, idx_v, *rest):
    bufs = rest[:_NBUF]
    gsems = rest[_NBUF:2 * _NBUF]
    ssems = rest[2 * _NBUF:3 * _NBUF]
    wid = lax.axis_index("s") * _NC + lax.axis_index("c")
    base = wid * _PER_W
    pltpu.sync_copy(tok_hbm.at[wid], idx_v)

    def start_gather(step, b):
        pltpu.async_copy(table_hbm.at[idx_v.at[step]], bufs[b], gsems[b])

    def scale(buf):
        def _mul_row(i, _):
            for k in range(_EMB // _L):
                sl = (i, pl.ds(k * _L, _L))
                buf[sl] = buf[sl] * _SCALE
            return 0

        lax.fori_loop(0, _CHUNK, _mul_row, 0)

    for j in range(_NBUF - 1):
        start_gather(j, j)
    scat = [None] * _NBUF
    for j in range(_NCHUNK):
        b = j % _NBUF
        ahead = j + _NBUF - 1
        if ahead < _NCHUNK:
            nb = ahead % _NBUF
            if scat[nb] is not None:
                scat[nb].wait()
            start_gather(ahead, nb)
        pltpu.make_async_copy(
            table_hbm.at[pl.ds(0, _CHUNK)], bufs[b], gsems[b]
        ).wait()
        scale(bufs[b])
        scat[b] = pltpu.async_copy(
            bufs[b], out_hbm.at[pl.ds(base + j * _CHUNK, _CHUNK)], ssems[b]
        )
    for h in scat:
        if h is not None:
            h.wait()


@jax.jit
def kernel(tokens, table):
    tok = tokens.astype(jnp.int32).reshape(_NW, _NCHUNK, _CHUNK)
    mesh = plsc.VectorSubcoreMesh(core_axis_name="c", subcore_axis_name="s")
    out = pl.kernel(
        _body,
        out_type=jax.ShapeDtypeStruct((_N, _EMB), jnp.float32),
        mesh=mesh,
        scratch_types=(
            [pltpu.VMEM((_NCHUNK, _CHUNK), jnp.int32)]
            + [pltpu.VMEM((_CHUNK, _EMB), jnp.float32) for _ in range(_NBUF)]
            + [pltpu.SemaphoreType.DMA for _ in range(2 * _NBUF)]
        ),
    )(tok, table)
    return out.reshape(_SEQ, _BATCH, _EMB)


# speedup vs baseline: 1.0338x; 1.0045x over previous
import math

import jax
import jax.numpy as jnp
from jax import lax
from jax.experimental import pallas as pl
from jax.experimental.pallas import tpu as pltpu
from jax.experimental.pallas import tpu_sc as plsc

_SEQ, _BATCH, _EMB = 200, 1024, 128
_N = _SEQ * _BATCH
_NC, _NS, _L = 2, 16, 16
_NW = _NC * _NS
_PER_W = _N // _NW
_CHUNK = 128
_NCHUNK = _PER_W // _CHUNK
_NBUF = 7
_K = 4
_SCALE = math.sqrt(_EMB)


def _body(tok_hbm, table_hbm, out_hbm, idx_v, *rest):
    bufs = rest[:_NBUF]
    gsems = rest[_NBUF:2 * _NBUF]
    ssems = rest[2 * _NBUF:3 * _NBUF]
    wid = lax.axis_index("s") * _NC + lax.axis_index("c")
    base = wid * _PER_W
    pltpu.sync_copy(tok_hbm.at[wid], idx_v)

    def start_gather(step, b):
        pltpu.async_copy(table_hbm.at[idx_v.at[step]], bufs[b], gsems[b])

    def scale(buf):
        def _mul_row(i, _):
            for k in range(_EMB // _L):
                sl = (i, pl.ds(k * _L, _L))
                buf[sl] = buf[sl] * _SCALE
            return 0

        lax.fori_loop(0, _CHUNK, _mul_row, 0)

    for j in range(_K):
        start_gather(j, j % _NBUF)
    for j in range(_NCHUNK):
        b = j % _NBUF
        ahead = j + _K
        if ahead < _NCHUNK:
            nb = ahead % _NBUF
            if ahead - _NBUF >= 0:
                pltpu.make_async_copy(
                    table_hbm.at[pl.ds(0, _CHUNK)], bufs[nb], ssems[nb]
                ).wait()
            start_gather(ahead, nb)
        pltpu.make_async_copy(
            table_hbm.at[pl.ds(0, _CHUNK)], bufs[b], gsems[b]
        ).wait()
        scale(bufs[b])
        pltpu.async_copy(
            bufs[b], out_hbm.at[pl.ds(base + j * _CHUNK, _CHUNK)], ssems[b]
        )
    for b in range(_NBUF):
        pltpu.make_async_copy(
            table_hbm.at[pl.ds(0, _CHUNK)], bufs[b], ssems[b]
        ).wait()


@jax.jit
def kernel(tokens, table):
    tok = tokens.astype(jnp.int32).reshape(_NW, _NCHUNK, _CHUNK)
    mesh = plsc.VectorSubcoreMesh(core_axis_name="c", subcore_axis_name="s")
    out = pl.kernel(
        _body,
        out_type=jax.ShapeDtypeStruct((_N, _EMB), jnp.float32),
        mesh=mesh,
        scratch_types=(
            [pltpu.VMEM((_NCHUNK, _CHUNK), jnp.int32)]
            + [pltpu.VMEM((_CHUNK, _EMB), jnp.float32) for _ in range(_NBUF)]
            + [pltpu.SemaphoreType.DMA for _ in range(2 * _NBUF)]
        ),
    )(tok, table)
    return out.reshape(_SEQ, _BATCH, _EMB)


# speedup vs baseline: 1.0509x; 1.0166x over previous
import math

import jax
import jax.numpy as jnp
from jax import lax
from jax.experimental import pallas as pl
from jax.experimental.pallas import tpu as pltpu
from jax.experimental.pallas import tpu_sc as plsc

_SEQ, _BATCH, _EMB = 200, 1024, 128
_N = _SEQ * _BATCH
_NC, _NS, _L = 2, 16, 16
_NW = _NC * _NS
_PER_W = _N // _NW
_CHUNK = 128
_NCHUNK = _PER_W // _CHUNK
_NBUF = 5
_K = 3
_NSS = _NCHUNK // _NBUF
_SCALE = math.sqrt(_EMB)


def _body(tok_hbm, table_hbm, out_hbm, idx_v, *rest):
    bufs = rest[:_NBUF]
    gsems = rest[_NBUF:2 * _NBUF]
    ssems = rest[2 * _NBUF:3 * _NBUF]
    wid = lax.axis_index("s") * _NC + lax.axis_index("c")
    base = wid * _PER_W
    pltpu.sync_copy(tok_hbm.at[wid], idx_v)

    def start_gather(step, b):
        pltpu.async_copy(table_hbm.at[idx_v.at[step]], bufs[b], gsems[b])

    def scale(buf):
        def _mul_row(i, _):
            for k in range(_EMB // _L):
                sl = (i, pl.ds(k * _L, _L))
                buf[sl] = buf[sl] * _SCALE
            return 0

        lax.fori_loop(0, _CHUNK, _mul_row, 0)

    def chunk(j, u, wait_wb, prefetch):
        b = u % _NBUF
        nb = (u + _K) % _NBUF
        if prefetch:
            if wait_wb:
                pltpu.make_async_copy(
                    table_hbm.at[pl.ds(0, _CHUNK)], bufs[nb], ssems[nb]
                ).wait()
            start_gather(j + _K, nb)
        pltpu.make_async_copy(
            table_hbm.at[pl.ds(0, _CHUNK)], bufs[b], gsems[b]
        ).wait()
        scale(bufs[b])
        pltpu.async_copy(
            bufs[b], out_hbm.at[pl.ds(base + j * _CHUNK, _CHUNK)], ssems[b]
        )

    for j in range(_K):
        start_gather(j, j)
    for u in range(_NBUF):
        chunk(u, u, wait_wb=(u >= _NBUF - _K), prefetch=True)

    def super_step(g, _):
        for u in range(_NBUF):
            chunk(g * _NBUF + u, u, wait_wb=True, prefetch=True)
        return 0

    lax.fori_loop(1, _NSS - 1, super_step, 0)
    for u in range(_NBUF):
        j = (_NSS - 1) * _NBUF + u
        chunk(j, u, wait_wb=True, prefetch=(j + _K < _NCHUNK))
    for b in range(_NBUF):
        pltpu.make_async_copy(
            table_hbm.at[pl.ds(0, _CHUNK)], bufs[b], ssems[b]
        ).wait()


@jax.jit
def kernel(tokens, table):
    tok = tokens.astype(jnp.int32).reshape(_NW, _NCHUNK, _CHUNK)
    mesh = plsc.VectorSubcoreMesh(core_axis_name="c", subcore_axis_name="s")
    out = pl.kernel(
        _body,
        out_type=jax.ShapeDtypeStruct((_N, _EMB), jnp.float32),
        mesh=mesh,
        scratch_types=(
            [pltpu.VMEM((_NCHUNK, _CHUNK), jnp.int32)]
            + [pltpu.VMEM((_CHUNK, _EMB), jnp.float32) for _ in range(_NBUF)]
            + [pltpu.SemaphoreType.DMA for _ in range(2 * _NBUF)]
        ),
    )(tok, table)
    return out.reshape(_SEQ, _BATCH, _EMB)
